# Initial kernel scaffold; baseline (speedup 1.0000x reference)
import jax
import jax.numpy as jnp
from jax.experimental import pallas as pl


def kernel(center_words, target_words, outer_words, emb_v, emb_u):
    raise NotImplementedError("write your pallas kernel here")


# speedup vs baseline: 5.1193x; 5.1193x over previous
import functools

import jax
import jax.numpy as jnp
from jax import lax
from jax.experimental import pallas as pl
from jax.experimental.pallas import tpu as pltpu
from jax.experimental.pallas import tpu_sc as plsc

VOCAB = 1000000
DIM = 64
BATCH = 16384
T = 20

NC = 2
NS = 16
LANES = 16
NW = NC * NS
CPW = BATCH // NW
CHUNK_C = 32
NCHUNK = CPW // CHUNK_C
ROWS = CHUNK_C * T
IDX_W = 128
NIDX = ROWS // IDX_W
SUB_C = 4
NSUB = CHUNK_C // SUB_C
SUB_ROWS = SUB_C * T


def _log_sigmoid(x):
    z = jnp.exp(-jnp.abs(x))
    y = z / (2.0 + z)
    y2 = y * y
    p = 1.0 + y2 * (
        (1.0 / 3.0) + y2 * ((1.0 / 5.0) + y2 * ((1.0 / 7.0) + y2 * (1.0 / 9.0)))
    )
    return jnp.minimum(x, 0.0) - 2.0 * y * p


def _row_sums_16(pbuf):
    base = lax.iota(jnp.int32, LANES) * LANES
    acc = plsc.load_gather(pbuf, [base])
    for l in range(1, LANES):
        acc = acc + plsc.load_gather(pbuf, [base + l])
    return acc


def _sc_body(cidx_hbm, tidx_hbm, emb_v, emb_u, out_hbm,
             cidx_v, tidx_v, cbuf, tbuf, pbuf, acc_pos, acc_neg, sem):
    wid = lax.axis_index("s") * NC + lax.axis_index("c")
    acc_pos[...] = jnp.zeros((LANES,), jnp.float32)
    acc_neg[...] = jnp.zeros((LANES,), jnp.float32)

    def chunk_body(g, carry):
        base_c = wid * CPW + g * CHUNK_C
        base_t = base_c * T
        pltpu.sync_copy(cidx_hbm.at[pl.ds(base_c, CHUNK_C)], cidx_v)
        pltpu.sync_copy(tidx_hbm.at[pl.ds(base_t, ROWS)], tidx_v)
        copies = [pltpu.async_copy(emb_v.at[cidx_v], cbuf, sem)]
        for i in range(NIDX):
            copies.append(
                pltpu.async_copy(emb_u.at[tidx_v.at[pl.ds(i * IDX_W, IDX_W)]],
                                 tbuf.at[pl.ds(i * IDX_W, IDX_W)], sem))
        for c in copies:
            c.wait()

        def sub_body(s, _):
            ap = acc_pos[...]
            for ci in range(SUB_C):
                crow = s * SUB_C + ci
                cvec = [cbuf[crow, pl.ds(k * LANES, LANES)] for k in range(4)]
                for t in range(T):
                    lrow = ci * T + t
                    r = s * SUB_ROWS + lrow
                    part = tbuf[r, pl.ds(0, LANES)] * cvec[0]
                    for k in range(1, 4):
                        part = part + tbuf[r, pl.ds(k * LANES, LANES)] * cvec[k]
                    j = lrow % LANES
                    pbuf[pl.ds(j * LANES, LANES)] = part
                    if lrow % LANES == LANES - 1:
                        ap = ap + _log_sigmoid(_row_sums_16(pbuf))
            acc_pos[...] = ap
            return 0

        lax.fori_loop(0, NSUB, sub_body, 0, unroll=1)

        an = acc_neg[...]
        for ci in range(CHUNK_C):
            cv = [cbuf[ci, pl.ds(k * LANES, LANES)] for k in range(4)]
            part = cv[0] * cv[0]
            for k in range(1, 4):
                part = part + cv[k] * cv[k]
            pbuf[pl.ds((ci % LANES) * LANES, LANES)] = part
            if ci % LANES == LANES - 1:
                an = an + _log_sigmoid(_row_sums_16(pbuf))
        acc_neg[...] = an
        return 0

    lax.fori_loop(0, NCHUNK, chunk_body, 0, unroll=1)

    acc_pos[...] = acc_pos[...] + jnp.float32(T) * acc_neg[...]
    pltpu.sync_copy(acc_pos, out_hbm.at[wid])


@jax.jit
def _skipgram_loss(cidx, tidx2d, emb_v, emb_u):
    mesh = plsc.VectorSubcoreMesh(
        core_axis_name="c", subcore_axis_name="s",
        num_cores=NC, num_subcores=NS)
    parts = pl.kernel(
        _sc_body,
        out_type=jax.ShapeDtypeStruct((NW, LANES), jnp.float32),
        mesh=mesh,
        compiler_params=pltpu.CompilerParams(
            needs_layout_passes=False, use_tc_tiling_on_sc=False),
        scratch_types=[
            pltpu.VMEM((CHUNK_C,), jnp.int32),
            pltpu.VMEM((ROWS,), jnp.int32),
            pltpu.VMEM((CHUNK_C, DIM), jnp.float32),
            pltpu.VMEM((ROWS, DIM), jnp.float32),
            pltpu.VMEM((LANES * LANES,), jnp.float32),
            pltpu.VMEM((LANES,), jnp.float32),
            pltpu.VMEM((LANES,), jnp.float32),
            pltpu.SemaphoreType.DMA,
        ],
    )(cidx, tidx2d, emb_v, emb_u)
    return -(jnp.sum(parts) / jnp.float32(BATCH * T))


def kernel(center_words, target_words, outer_words, emb_v, emb_u):
    del outer_words
    cidx = center_words.reshape(BATCH)
    tidx = target_words.reshape(BATCH * T)
    return _skipgram_loss(cidx, tidx, emb_v, emb_u)
